# baseline (speedup 1.0000x reference)
import functools

import jax
import jax.numpy as jnp
from jax import lax
from jax.experimental import pallas as pl
from jax.experimental.pallas import tpu as pltpu
from jax.experimental.pallas import tpu_sc as plsc


_NUM_WORKERS = 32
_GATHER_WINDOW = 32
_BM = 1024


def _sc_gather(table, idx):
    n_tok = idx.shape[0]
    dim = table.shape[1]
    b_per_w = n_tok // _NUM_WORKERS
    ch = _GATHER_WINDOW
    n_ch = b_per_w // ch
    mesh = plsc.VectorSubcoreMesh(core_axis_name="c", subcore_axis_name="s")

    @functools.partial(
        pl.kernel,
        out_type=jax.ShapeDtypeStruct((n_tok, dim), table.dtype),
        mesh=mesh,
        scratch_types=[
            pltpu.VMEM((b_per_w,), jnp.int32),
            pltpu.VMEM((3, ch, dim), jnp.float32),
            pltpu.SemaphoreType.DMA,
            pltpu.SemaphoreType.DMA,
            pltpu.SemaphoreType.DMA,
            pltpu.SemaphoreType.DMA,
            pltpu.SemaphoreType.DMA,
            pltpu.SemaphoreType.DMA,
            pltpu.SemaphoreType.DMA,
        ],
    )
    def gather_kernel(table_hbm, idx_hbm, out_hbm, idx_v, rows_v, sem_i,
                      sg0, sg1, sg2, so0, so1, so2):
        wid = lax.axis_index("s") * 2 + lax.axis_index("c")
        base = wid * b_per_w
        pltpu.async_copy(idx_hbm.at[pl.ds(base, b_per_w)], idx_v, sem_i).wait()
        sg = (sg0, sg1, sg2)
        so = (so0, so1, so2)

        def g_copy(c):
            s = c % 3
            return pltpu.make_async_copy(
                table_hbm.at[idx_v.at[pl.ds(c * ch, ch)]], rows_v.at[s], sg[s])

        def o_copy(c):
            s = c % 3
            return pltpu.make_async_copy(
                rows_v.at[s], out_hbm.at[pl.ds(base + c * ch, ch)], so[s])

        for c in range(n_ch):
            if c >= 3:
                o_copy(c - 3).wait()
            g_copy(c).start()
            if c >= 1:
                g_copy(c - 1).wait()
                o_copy(c - 1).start()
        last = n_ch - 1
        for c in range(max(last - 2, 0), last):
            o_copy(c).wait()
        g_copy(last).wait()
        o_copy(last).start()
        o_copy(last).wait()

    return gather_kernel(table, idx)


def _mm_body(a_ref, w_ref, b_ref, o_ref):
    o_ref[...] = (
        jnp.dot(a_ref[...], w_ref[...], preferred_element_type=jnp.float32)
        + b_ref[...]
    )


def _tc_matmul(a, w, b2d):
    m, k = a.shape
    n = w.shape[1]
    return pl.pallas_call(
        _mm_body,
        grid=(m // _BM,),
        in_specs=[
            pl.BlockSpec((_BM, k), lambda i: (i, 0)),
            pl.BlockSpec((k, n), lambda i: (0, 0)),
            pl.BlockSpec((1, n), lambda i: (0, 0)),
        ],
        out_specs=pl.BlockSpec((_BM, n), lambda i: (i, 0)),
        out_shape=jax.ShapeDtypeStruct((m, n), jnp.float32),
    )(a, w, b2d)


def kernel(text_tokens, emb_table, proj_w, proj_b):
    bsz, seq = text_tokens.shape
    n_out = proj_w.shape[1]
    tokens = text_tokens.reshape(-1).astype(jnp.int32)
    gathered = _sc_gather(emb_table, tokens)
    out = _tc_matmul(gathered, proj_w, proj_b.reshape(1, n_out))
    return out.reshape(bsz, seq, n_out)
